# Initial kernel scaffold; baseline (speedup 1.0000x reference)
import jax
import jax.numpy as jnp
from jax.experimental import pallas as pl


def kernel(n_feat, e_feat, edge_index, graph_ids, W_i, b_i, W_h, b_h, W_o, b_o, gate_W, gate_b, proj_W, proj_b):
    raise NotImplementedError("write your pallas kernel here")


# speedup vs baseline: 1.0359x; 1.0359x over previous
import functools

import jax
import jax.numpy as jnp
from jax.experimental import pallas as pl

_N = 50000
_G = 512
_H = 128
_T = 12
_STEPS = 4


def _mm_bias_kernel(a_ref, b_ref, bias_ref, o_ref, *, relu):
    acc = jnp.dot(a_ref[...], b_ref[...], preferred_element_type=jnp.float32)
    acc = acc + bias_ref[...]
    if relu:
        acc = jnp.maximum(acc, 0.0)
    o_ref[...] = acc


def _matmul_bias(a, b, bias, relu=False, bm=400):
    M, K = a.shape
    H = b.shape[1]
    assert M % bm == 0, (M, bm)
    return pl.pallas_call(
        functools.partial(_mm_bias_kernel, relu=relu),
        grid=(M // bm,),
        in_specs=[
            pl.BlockSpec((bm, K), lambda i: (i, 0)),
            pl.BlockSpec((K, H), lambda i: (0, 0)),
            pl.BlockSpec((1, H), lambda i: (0, 0)),
        ],
        out_specs=pl.BlockSpec((bm, H), lambda i: (i, 0)),
        out_shape=jax.ShapeDtypeStruct((M, H), jnp.float32),
    )(a, b, bias.reshape(1, H))


def kernel(n_feat, e_feat, edge_index, graph_ids, W_i, b_i, W_h, b_h,
           W_o, b_o, gate_W, gate_b, proj_W, proj_b):
    src = edge_index[0]
    dst = edge_index[1]
    D_IN = n_feat.shape[1]

    p = _matmul_bias(n_feat, W_i[:D_IN], b_i, relu=False)
    q = e_feat @ W_i[D_IN:]
    h0 = jnp.maximum(p[src] + q, 0.0)

    A = jax.ops.segment_sum(h0, dst, num_segments=_N)
    for _ in range(_STEPS):
        m = _matmul_bias(A, W_h, b_h, relu=False)
        A = jax.ops.segment_sum(jnp.maximum(h0 + m[src], 0.0), dst,
                                num_segments=_N)

    hv = jnp.maximum(n_feat @ W_o[:D_IN] + _matmul_bias(A, W_o[D_IN:], b_o),
                     0.0)
    logits = hv @ gate_W.T + gate_b

    seg_max = jax.ops.segment_max(logits, graph_ids, num_segments=_G)
    a = jnp.exp(logits - seg_max[graph_ids])
    denom = jax.ops.segment_sum(a, graph_ids, num_segments=_G)
    S = jnp.stack(
        [jax.ops.segment_sum(a[:, t:t + 1] * hv, graph_ids, num_segments=_G)
         for t in range(_T)], axis=1)
    inv = 1.0 / (denom + 1e-9)
    S = S * inv[:, :, None]
    c = denom * inv

    out = jnp.einsum('gth,thd->gtd', S, proj_W) + c[:, :, None] * proj_b[None]
    return out
